# Initial kernel scaffold; baseline (speedup 1.0000x reference)
import jax
import jax.numpy as jnp
from jax.experimental import pallas as pl


def kernel(x, segment_info, token_table, segment_table, pos_emb):
    raise NotImplementedError("write your pallas kernel here")


# speedup vs baseline: 6.4339x; 6.4339x over previous
import functools

import jax
import jax.numpy as jnp
from jax import lax
from jax.experimental import pallas as pl
from jax.experimental.pallas import tpu as pltpu, tpu_sc as plsc

HIDDEN = 128
SEQ = 200
NSEG = 3
LANES = 16
NC, NS = 2, 16
NW = NC * NS
CHUNK = 128


def _sc_embedding_call(n_rows, vocab):
    rows_per_w = n_rows // NW
    n_chunks = rows_per_w // CHUNK
    mesh = plsc.VectorSubcoreMesh(core_axis_name="c", subcore_axis_name="s",
                                  num_cores=NC, num_subcores=NS)

    @functools.partial(
        pl.kernel,
        out_type=jax.ShapeDtypeStruct((n_rows, HIDDEN), jnp.float32),
        mesh=mesh,
        scratch_types=[
            pltpu.VMEM((rows_per_w,), jnp.int32),
            pltpu.VMEM((rows_per_w,), jnp.int32),
            pltpu.VMEM((rows_per_w,), jnp.int32),
            pltpu.VMEM((CHUNK, HIDDEN), jnp.float32),
            pltpu.VMEM((CHUNK, HIDDEN), jnp.float32),
            pltpu.SemaphoreType.DMA,
            pltpu.SemaphoreType.DMA,
        ],
    )
    def call(x_hbm, seg_hbm, ttab_hbm, ctab_hbm, out_hbm,
             tok_v, seg_v, cidx_v, buf_a, buf_b, sem_a, sem_b):
        wid = lax.axis_index("s") * NC + lax.axis_index("c")
        base = wid * rows_per_w

        pltpu.sync_copy(x_hbm.at[pl.ds(base, rows_per_w)], tok_v)
        pltpu.sync_copy(seg_hbm.at[pl.ds(base, rows_per_w)], seg_v)

        lane = lax.iota(jnp.int32, LANES)

        def cidx_body(k, _):
            off = k * LANES
            s16 = seg_v[pl.ds(off, LANES)]
            pos = lax.rem(off + lane, SEQ)
            cidx_v[pl.ds(off, LANES)] = s16 * SEQ + pos
            return _

        lax.fori_loop(0, rows_per_w // LANES, cidx_body, None)

        def chunk_body(j, _):
            row = j * CHUNK
            cp_a = pltpu.async_copy(
                ctab_hbm.at[cidx_v.at[pl.ds(row, CHUNK)]], buf_a, sem_a)
            cp_b = pltpu.async_copy(
                ttab_hbm.at[tok_v.at[pl.ds(row, CHUNK)]], buf_b, sem_b)
            cp_a.wait()
            cp_b.wait()

            def add_row(i, _):
                for c in range(HIDDEN // LANES):
                    sl = pl.ds(c * LANES, LANES)
                    buf_a[i, sl] = buf_a[i, sl] + buf_b[i, sl]
                return _

            lax.fori_loop(0, CHUNK, add_row, None)
            pltpu.sync_copy(buf_a, out_hbm.at[pl.ds(base + row, CHUNK)])
            return _

        lax.fori_loop(0, n_chunks, chunk_body, None)

    return call


def kernel(x, segment_info, token_table, segment_table, pos_emb):
    batch, seq = x.shape
    n_rows = batch * seq
    x_flat = x.reshape(n_rows).astype(jnp.int32)
    seg_flat = segment_info.reshape(n_rows).astype(jnp.int32)
    ctab = (segment_table[:, None, :] + pos_emb[None, :, :]).reshape(
        NSEG * SEQ, HIDDEN)
    call = _sc_embedding_call(n_rows, token_table.shape[0])
    out = call(x_flat, seg_flat, token_table, ctab)
    return out.reshape(batch, seq, HIDDEN)


# speedup vs baseline: 7.7392x; 1.2029x over previous
import functools

import jax
import jax.numpy as jnp
from jax import lax
from jax.experimental import pallas as pl
from jax.experimental.pallas import tpu as pltpu, tpu_sc as plsc

HIDDEN = 128
SEQ = 200
NSEG = 3
LANES = 16
NC, NS = 2, 16
NW = NC * NS
CHUNK = 128


def _sc_embedding_call(n_rows, vocab):
    rows_per_w = n_rows // NW
    n_chunks = rows_per_w // CHUNK
    mesh = plsc.VectorSubcoreMesh(core_axis_name="c", subcore_axis_name="s",
                                  num_cores=NC, num_subcores=NS)

    nbuf = 2
    assert n_chunks % nbuf == 0 and n_chunks >= 2 * nbuf

    @functools.partial(
        pl.kernel,
        out_type=jax.ShapeDtypeStruct((n_rows, HIDDEN), jnp.float32),
        mesh=mesh,
        scratch_types=[
            pltpu.VMEM((rows_per_w,), jnp.int32),
            pltpu.VMEM((rows_per_w,), jnp.int32),
            pltpu.VMEM((rows_per_w,), jnp.int32),
            [pltpu.VMEM((CHUNK, HIDDEN), jnp.float32) for _ in range(nbuf)],
            [pltpu.VMEM((CHUNK, HIDDEN), jnp.float32) for _ in range(nbuf)],
            [pltpu.VMEM((CHUNK, HIDDEN), jnp.float32) for _ in range(nbuf)],
            [pltpu.SemaphoreType.DMA for _ in range(nbuf)],
            [pltpu.SemaphoreType.DMA for _ in range(nbuf)],
        ],
    )
    def call(x_hbm, seg_hbm, ttab_hbm, ctab_hbm, out_hbm,
             tok_v, seg_v, cidx_v, buf_a, buf_b, buf_r, sem_g, sem_s):
        wid = lax.axis_index("s") * NC + lax.axis_index("c")
        base = wid * rows_per_w

        pltpu.sync_copy(x_hbm.at[pl.ds(base, rows_per_w)], tok_v)
        pltpu.sync_copy(seg_hbm.at[pl.ds(base, rows_per_w)], seg_v)

        lane = lax.iota(jnp.int32, LANES)

        def cidx_body(k, _):
            off = k * LANES
            s16 = seg_v[pl.ds(off, LANES)]
            pos = lax.rem(off + lane, SEQ)
            cidx_v[pl.ds(off, LANES)] = s16 * SEQ + pos
            return _

        lax.fori_loop(0, rows_per_w // LANES, cidx_body, None)

        def fire_gathers(chunk, b):
            row = chunk * CHUNK
            pltpu.async_copy(
                ctab_hbm.at[cidx_v.at[pl.ds(row, CHUNK)]], buf_a[b], sem_g[b])
            pltpu.async_copy(
                ttab_hbm.at[tok_v.at[pl.ds(row, CHUNK)]], buf_b[b], sem_g[b])

        def drain_gathers(b):
            pltpu.make_async_copy(
                ctab_hbm.at[cidx_v.at[pl.ds(0, CHUNK)]], buf_a[b],
                sem_g[b]).wait()
            pltpu.make_async_copy(
                ttab_hbm.at[tok_v.at[pl.ds(0, CHUNK)]], buf_b[b],
                sem_g[b]).wait()

        for b in range(nbuf):
            fire_gathers(b, b)

        def outer(g, _):
            for b in range(nbuf):
                chunk = g * nbuf + b
                drain_gathers(b)

                @pl.when(chunk >= nbuf)
                def _():
                    pltpu.make_async_copy(
                        buf_r[b], out_hbm.at[pl.ds(base, CHUNK)],
                        sem_s[b]).wait()

                def add_row(i, _i):
                    for c in range(HIDDEN // LANES):
                        sl = pl.ds(c * LANES, LANES)
                        buf_r[b][i, sl] = buf_a[b][i, sl] + buf_b[b][i, sl]
                    return _i

                lax.fori_loop(0, CHUNK, add_row, None)

                @pl.when(chunk + nbuf < n_chunks)
                def _():
                    fire_gathers(chunk + nbuf, b)

                pltpu.async_copy(
                    buf_r[b], out_hbm.at[pl.ds(base + chunk * CHUNK, CHUNK)],
                    sem_s[b])
            return _

        lax.fori_loop(0, n_chunks // nbuf, outer, None)
        for b in range(nbuf):
            pltpu.make_async_copy(
                buf_r[b], out_hbm.at[pl.ds(base, CHUNK)], sem_s[b]).wait()

    return call


def kernel(x, segment_info, token_table, segment_table, pos_emb):
    batch, seq = x.shape
    n_rows = batch * seq
    x_flat = x.reshape(n_rows).astype(jnp.int32)
    seg_flat = segment_info.reshape(n_rows).astype(jnp.int32)
    ctab = (segment_table[:, None, :] + pos_emb[None, :, :]).reshape(
        NSEG * SEQ, HIDDEN)
    call = _sc_embedding_call(n_rows, token_table.shape[0])
    out = call(x_flat, seg_flat, token_table, ctab)
    return out.reshape(batch, seq, HIDDEN)
